# Initial kernel scaffold; baseline (speedup 1.0000x reference)
import jax
import jax.numpy as jnp
from jax.experimental import pallas as pl


def kernel(x, embed_pool, W1, b1, W2, b2, W3, b3, D1, d1, D2, d2, D3, d3):
    raise NotImplementedError("write your pallas kernel here")


# speedup vs baseline: 1.1741x; 1.1741x over previous
import jax
import jax.numpy as jnp
from jax.experimental import pallas as pl
from jax.experimental.pallas import tpu as pltpu

_B = 8192
_IN = 768
_K = 1024
_D = 64
_H = 512
_BM = 1024
_GRID = _B // _BM
_INV_SQRT2 = 0.7071067811865476


def _gelu_erf(v):
    return 0.5 * v * (1.0 + jax.lax.erf(v * _INV_SQRT2))


def _dot(a, b, dims, precision=None):
    return jax.lax.dot_general(a, b, (dims, ((), ())),
                               preferred_element_type=jnp.float32,
                               precision=precision)


def _mm_bias_body(a_ref, w_ref, b_ref, o_ref):
    o_ref[:, :] = _dot(a_ref[:, :], w_ref[:, :], ((1,), (0,))) + b_ref[:, :]


def _mm_bias(a, w, b):
    m, k = a.shape
    n = w.shape[1]
    return pl.pallas_call(
        _mm_bias_body,
        grid=(_GRID,),
        in_specs=[pl.BlockSpec((m // _GRID, k), lambda i: (i, 0)),
                  pl.BlockSpec((k, n), lambda i: (0, 0)),
                  pl.BlockSpec((1, n), lambda i: (0, 0))],
        out_specs=pl.BlockSpec((m // _GRID, n), lambda i: (i, 0)),
        out_shape=jax.ShapeDtypeStruct((m, n), jnp.float32),
    )(a, w, b.reshape(1, n))


def _vq_dec_body(x_ref, z_ref, zsq_ref, esq_ref, e_ref, d1w_ref, d1b_ref,
                 d2w_ref, d2b_ref, d3w_ref, d3b_ref, xp_ref, zd_ref, acc_ref):
    i = pl.program_id(0)

    z_e = z_ref[:, :]
    e = e_ref[:, :]
    cross = _dot(z_e, e, ((1,), (1,)))
    d2m = zsq_ref[:, :] + esq_ref[:, :] - 2.0 * cross
    factor = jnp.sqrt(jnp.maximum(d2m, 0.0))

    col = jax.lax.broadcasted_iota(jnp.int32, (_BM, _K), 1)
    fmin = jnp.min(factor, axis=1, keepdims=True)
    idx = jnp.min(jnp.where(factor == fmin, col, _K), axis=1, keepdims=True)
    onehot = col == idx
    zd_ref[:, :] = onehot.astype(jnp.int32)
    z_q = _dot(onehot.astype(jnp.float32), e, ((1,), (0,)),
               precision=jax.lax.Precision.HIGHEST)

    g = _gelu_erf(_dot(z_q, d1w_ref[:, :], ((1,), (0,))) + d1b_ref[:, :])
    g = _gelu_erf(_dot(g, d2w_ref[:, :], ((1,), (0,))) + d2b_ref[:, :])
    xp = jax.nn.sigmoid(_dot(g, d3w_ref[:, :], ((1,), (0,))) + d3b_ref[:, :])
    xp_ref[:, :] = xp

    diff = x_ref[:, :] - xp
    vqd = z_e - z_q

    @pl.when(i == 0)
    def _init():
        acc_ref[0, 0] = 0.0
        acc_ref[0, 1] = 0.0

    acc_ref[0, 0] += jnp.sum(diff * diff)
    acc_ref[0, 1] += jnp.sum(vqd * vqd)


def kernel(x, embed_pool, W1, b1, W2, b2, W3, b3, D1, d1, D2, d2, D3, d3):
    h = jax.nn.gelu(_mm_bias(x, W1, b1), approximate=False)
    h = jax.nn.gelu(_mm_bias(h, W2, b2), approximate=False)
    z_e = _mm_bias(h, W3, b3)
    z_sq = jnp.sum(z_e**2, axis=1, keepdims=True)
    e_sq = jnp.sum(embed_pool**2, axis=1)[None, :]

    def full(shape):
        return pl.BlockSpec(shape, lambda i: (0, 0))

    xp, zd, acc = pl.pallas_call(
        _vq_dec_body,
        grid=(_GRID,),
        in_specs=[
            pl.BlockSpec((_BM, _IN), lambda i: (i, 0)),
            pl.BlockSpec((_BM, _D), lambda i: (i, 0)),
            pl.BlockSpec((_BM, 1), lambda i: (i, 0)),
            full((1, _K)),
            full((_K, _D)),
            full((_D, _H)), full((1, _H)),
            full((_H, _H)), full((1, _H)),
            full((_H, _IN)), full((1, _IN)),
        ],
        out_specs=[
            pl.BlockSpec((_BM, _IN), lambda i: (i, 0)),
            pl.BlockSpec((_BM, _K), lambda i: (i, 0)),
            pl.BlockSpec(memory_space=pltpu.SMEM),
        ],
        out_shape=[
            jax.ShapeDtypeStruct((_B, _IN), jnp.float32),
            jax.ShapeDtypeStruct((_B, _K), jnp.int32),
            jax.ShapeDtypeStruct((1, 2), jnp.float32),
        ],
        compiler_params=pltpu.CompilerParams(
            dimension_semantics=("arbitrary",)),
    )(x, z_e, z_sq, e_sq, embed_pool, D1, d1.reshape(1, _H), D2,
      d2.reshape(1, _H), D3, d3.reshape(1, _IN))

    loss = (acc[0, 0] / (_B * _IN) + 1.25 * acc[0, 1] / (_B * _D)) / _B
    return xp, zd, loss


# speedup vs baseline: 1.2174x; 1.0369x over previous
import jax
import jax.numpy as jnp
from jax.experimental import pallas as pl
from jax.experimental.pallas import tpu as pltpu

_B = 8192
_IN = 768
_K = 1024
_D = 64
_H = 512
_BM = 1024
_GRID = _B // _BM
_INV_SQRT2 = 0.7071067811865476


def _gelu_erf(v):
    return 0.5 * v * (1.0 + jax.lax.erf(v * _INV_SQRT2))


def _dot(a, b, dims, precision=None):
    return jax.lax.dot_general(a, b, (dims, ((), ())),
                               preferred_element_type=jnp.float32,
                               precision=precision)


def _mm_bias_body(a_ref, w_ref, b_ref, o_ref):
    o_ref[:, :] = _dot(a_ref[:, :], w_ref[:, :], ((1,), (0,))) + b_ref[:, :]


def _mm_bias(a, w, b):
    m, k = a.shape
    n = w.shape[1]
    return pl.pallas_call(
        _mm_bias_body,
        grid=(_GRID,),
        in_specs=[pl.BlockSpec((m // _GRID, k), lambda i: (i, 0)),
                  pl.BlockSpec((k, n), lambda i: (0, 0)),
                  pl.BlockSpec((1, n), lambda i: (0, 0))],
        out_specs=pl.BlockSpec((m // _GRID, n), lambda i: (i, 0)),
        out_shape=jax.ShapeDtypeStruct((m, n), jnp.float32),
    )(a, w, b.reshape(1, n))


def _vq_dec_body(x_ref, h_ref, w3_ref, b3_ref, esq_ref, e_ref, d1w_ref,
                 d1b_ref, d2w_ref, d2b_ref, d3w_ref, d3b_ref, xp_ref, zd_ref,
                 acc_ref):
    i = pl.program_id(0)

    z_e = _dot(h_ref[:, :], w3_ref[:, :], ((1,), (0,))) + b3_ref[:, :]
    zsq = jnp.sum(z_e * z_e, axis=1, keepdims=True)
    e = e_ref[:, :]
    cross = _dot(z_e, e, ((1,), (1,)))
    d2m = zsq + esq_ref[:, :] - 2.0 * cross
    factor = jnp.sqrt(jnp.maximum(d2m, 0.0))

    col = jax.lax.broadcasted_iota(jnp.int32, (_BM, _K), 1)
    fmin = jnp.min(factor, axis=1, keepdims=True)
    idx = jnp.min(jnp.where(factor == fmin, col, _K), axis=1, keepdims=True)
    onehot = col == idx
    zd_ref[:, :] = onehot.astype(jnp.int32)
    z_q = _dot(onehot.astype(jnp.float32), e, ((1,), (0,)),
               precision=jax.lax.Precision.HIGHEST)

    g = _gelu_erf(_dot(z_q, d1w_ref[:, :], ((1,), (0,))) + d1b_ref[:, :])
    g = _gelu_erf(_dot(g, d2w_ref[:, :], ((1,), (0,))) + d2b_ref[:, :])
    xp = jax.nn.sigmoid(_dot(g, d3w_ref[:, :], ((1,), (0,))) + d3b_ref[:, :])
    xp_ref[:, :] = xp

    diff = x_ref[:, :] - xp
    vqd = z_e - z_q

    @pl.when(i == 0)
    def _init():
        acc_ref[0, 0] = 0.0
        acc_ref[0, 1] = 0.0

    acc_ref[0, 0] += jnp.sum(diff * diff)
    acc_ref[0, 1] += jnp.sum(vqd * vqd)


def kernel(x, embed_pool, W1, b1, W2, b2, W3, b3, D1, d1, D2, d2, D3, d3):
    h = jax.nn.gelu(_mm_bias(x, W1, b1), approximate=False)
    h = jax.nn.gelu(_mm_bias(h, W2, b2), approximate=False)
    e_sq = jnp.sum(embed_pool**2, axis=1)[None, :]

    def full(shape):
        return pl.BlockSpec(shape, lambda i: (0, 0))

    xp, zd, acc = pl.pallas_call(
        _vq_dec_body,
        grid=(_GRID,),
        in_specs=[
            pl.BlockSpec((_BM, _IN), lambda i: (i, 0)),
            pl.BlockSpec((_BM, _H), lambda i: (i, 0)),
            full((_H, _D)), full((1, _D)),
            full((1, _K)),
            full((_K, _D)),
            full((_D, _H)), full((1, _H)),
            full((_H, _H)), full((1, _H)),
            full((_H, _IN)), full((1, _IN)),
        ],
        out_specs=[
            pl.BlockSpec((_BM, _IN), lambda i: (i, 0)),
            pl.BlockSpec((_BM, _K), lambda i: (i, 0)),
            pl.BlockSpec(memory_space=pltpu.SMEM),
        ],
        out_shape=[
            jax.ShapeDtypeStruct((_B, _IN), jnp.float32),
            jax.ShapeDtypeStruct((_B, _K), jnp.int32),
            jax.ShapeDtypeStruct((1, 2), jnp.float32),
        ],
        compiler_params=pltpu.CompilerParams(
            dimension_semantics=("arbitrary",)),
    )(x, h, W3, b3.reshape(1, _D), e_sq, embed_pool, D1, d1.reshape(1, _H),
      D2, d2.reshape(1, _H), D3, d3.reshape(1, _IN))

    loss = (acc[0, 0] / (_B * _IN) + 1.25 * acc[0, 1] / (_B * _D)) / _B
    return xp, zd, loss


# speedup vs baseline: 1.3377x; 1.0988x over previous
import jax
import jax.numpy as jnp
from jax.experimental import pallas as pl
from jax.experimental.pallas import tpu as pltpu

_B = 8192
_IN = 768
_K = 1024
_D = 64
_H = 512
_BM = 1024
_GRID = _B // _BM
_INV_SQRT2 = 0.7071067811865476


def _gelu_exact(x):
    u = (-x) * 0.707106769
    abs_u = jnp.abs(u)
    u2 = u * u
    p = 7.85386146e-05 * u2 + (-0.000801019371)
    p = p * u2 + 0.00518832775
    p = p * u2 + (-0.0268538129)
    p = p * u2 + 0.112835854
    p = p * u2 + (-0.37612626)
    p = p * u2 + 1.12837911
    one_minus_erf = 1.0 - u * p
    neg_u2 = -u2
    z = jnp.exp(neg_u2)
    zq = z * (1.0 / abs_u)
    r = 1.0 / u2
    p1 = 0.0232682 * r + (-0.138703942)
    p1 = p1 * r + 0.368742466
    p1 = p1 * r + (-0.582473278)
    p1 = p1 * r + 0.621000469
    p1 = p1 * r + (-0.494451523)
    p1 = p1 * r + 0.340488
    p1 = p1 * r + (-0.274112701)
    p1 = p1 * r + 0.563825965
    p2 = (-10.477664) * r + 12.9772
    p2 = p2 * r + (-7.49551868)
    p2 = p2 * r + 2.92101908
    p2 = p2 * r + (-1.01526523)
    p2 = p2 * r + 0.42184633
    p2 = p2 * r + (-0.282076746)
    p2 = p2 * r + 0.564189494
    sel = jnp.where(abs_u < 2.0, p1, p2)
    big = zq * sel
    big = jnp.where(neg_u2 < -88.7228394, 0.0, big)
    big = jnp.where(u < 0.0, 2.0 - big, big)
    erfc_res = jnp.where(abs_u < 1.0, one_minus_erf, big)
    return (x * 0.5) * erfc_res


def _gelu_fast(v):
    return 0.5 * v * (1.0 + jax.lax.erf(v * _INV_SQRT2))


def _dot(a, b, dims, precision=None):
    return jax.lax.dot_general(a, b, (dims, ((), ())),
                               preferred_element_type=jnp.float32,
                               precision=precision)


def _fused_body(x_ref, e_ref, esq_ref, w1_ref, b1_ref, w2_ref, b2_ref,
                w3_ref, b3_ref, d1w_ref, d1b_ref, d2w_ref, d2b_ref, d3w_ref,
                d3b_ref, xp_ref, zd_ref, acc_ref):
    i = pl.program_id(0)

    x = x_ref[:, :]
    h = _gelu_exact(_dot(x, w1_ref[:, :], ((1,), (0,))) + b1_ref[:, :])
    h = _gelu_exact(_dot(h, w2_ref[:, :], ((1,), (0,))) + b2_ref[:, :])
    z_e = _dot(h, w3_ref[:, :], ((1,), (0,))) + b3_ref[:, :]

    zsq = jnp.sum(z_e * z_e, axis=1, keepdims=True)
    e = e_ref[:, :]
    cross = _dot(z_e, e, ((1,), (1,)))
    d2m = zsq + esq_ref[:, :] - 2.0 * cross
    factor = jnp.sqrt(jnp.maximum(d2m, 0.0))

    col = jax.lax.broadcasted_iota(jnp.int32, (_BM, _K), 1)
    fmin = jnp.min(factor, axis=1, keepdims=True)
    idx = jnp.min(jnp.where(factor == fmin, col, _K), axis=1, keepdims=True)
    onehot = col == idx
    zd_ref[:, :] = onehot.astype(jnp.int32)
    z_q = _dot(onehot.astype(jnp.float32), e, ((1,), (0,)),
               precision=jax.lax.Precision.HIGHEST)

    g = _gelu_fast(_dot(z_q, d1w_ref[:, :], ((1,), (0,))) + d1b_ref[:, :])
    g = _gelu_fast(_dot(g, d2w_ref[:, :], ((1,), (0,))) + d2b_ref[:, :])
    xp = jax.nn.sigmoid(_dot(g, d3w_ref[:, :], ((1,), (0,))) + d3b_ref[:, :])
    xp_ref[:, :] = xp

    diff = x - xp
    vqd = z_e - z_q

    @pl.when(i == 0)
    def _init():
        acc_ref[0, 0] = 0.0
        acc_ref[0, 1] = 0.0

    acc_ref[0, 0] += jnp.sum(diff * diff)
    acc_ref[0, 1] += jnp.sum(vqd * vqd)


def kernel(x, embed_pool, W1, b1, W2, b2, W3, b3, D1, d1, D2, d2, D3, d3):
    e_sq = jnp.sum(embed_pool**2, axis=1)[None, :]

    def full(shape):
        return pl.BlockSpec(shape, lambda i: (0, 0))

    xp, zd, acc = pl.pallas_call(
        _fused_body,
        grid=(_GRID,),
        in_specs=[
            pl.BlockSpec((_BM, _IN), lambda i: (i, 0)),
            full((_K, _D)),
            full((1, _K)),
            full((_IN, _H)), full((1, _H)),
            full((_H, _H)), full((1, _H)),
            full((_H, _D)), full((1, _D)),
            full((_D, _H)), full((1, _H)),
            full((_H, _H)), full((1, _H)),
            full((_H, _IN)), full((1, _IN)),
        ],
        out_specs=[
            pl.BlockSpec((_BM, _IN), lambda i: (i, 0)),
            pl.BlockSpec((_BM, _K), lambda i: (i, 0)),
            pl.BlockSpec(memory_space=pltpu.SMEM),
        ],
        out_shape=[
            jax.ShapeDtypeStruct((_B, _IN), jnp.float32),
            jax.ShapeDtypeStruct((_B, _K), jnp.int32),
            jax.ShapeDtypeStruct((1, 2), jnp.float32),
        ],
        compiler_params=pltpu.CompilerParams(
            dimension_semantics=("arbitrary",)),
    )(x, embed_pool, e_sq, W1, b1.reshape(1, _H), W2, b2.reshape(1, _H),
      W3, b3.reshape(1, _D), D1, d1.reshape(1, _H), D2, d2.reshape(1, _H),
      D3, d3.reshape(1, _IN))

    loss = (acc[0, 0] / (_B * _IN) + 1.25 * acc[0, 1] / (_B * _D)) / _B
    return xp, zd, loss


# speedup vs baseline: 1.3394x; 1.0012x over previous
import jax
import jax.numpy as jnp
from jax.experimental import pallas as pl
from jax.experimental.pallas import tpu as pltpu

_B = 8192
_IN = 768
_K = 1024
_D = 64
_H = 512
_BM = 1024
_GRID = _B // _BM
_INV_SQRT2 = 0.7071067811865476


def _gelu_exact(x):
    u = (-x) * 0.707106769
    abs_u = jnp.abs(u)
    u2 = u * u
    p = 7.85386146e-05 * u2 + (-0.000801019371)
    p = p * u2 + 0.00518832775
    p = p * u2 + (-0.0268538129)
    p = p * u2 + 0.112835854
    p = p * u2 + (-0.37612626)
    p = p * u2 + 1.12837911
    one_minus_erf = 1.0 - u * p
    neg_u2 = -u2
    z = jnp.exp(neg_u2)
    zq = z * (1.0 / abs_u)
    r = 1.0 / u2
    p1 = 0.0232682 * r + (-0.138703942)
    p1 = p1 * r + 0.368742466
    p1 = p1 * r + (-0.582473278)
    p1 = p1 * r + 0.621000469
    p1 = p1 * r + (-0.494451523)
    p1 = p1 * r + 0.340488
    p1 = p1 * r + (-0.274112701)
    p1 = p1 * r + 0.563825965
    p2 = (-10.477664) * r + 12.9772
    p2 = p2 * r + (-7.49551868)
    p2 = p2 * r + 2.92101908
    p2 = p2 * r + (-1.01526523)
    p2 = p2 * r + 0.42184633
    p2 = p2 * r + (-0.282076746)
    p2 = p2 * r + 0.564189494
    sel = jnp.where(abs_u < 2.0, p1, p2)
    big = zq * sel
    big = jnp.where(neg_u2 < -88.7228394, 0.0, big)
    big = jnp.where(u < 0.0, 2.0 - big, big)
    erfc_res = jnp.where(abs_u < 1.0, one_minus_erf, big)
    return (x * 0.5) * erfc_res


def _gelu_fast(v):
    return 0.5 * v * (1.0 + jax.lax.erf(v * _INV_SQRT2))


def _dot(a, b, dims, precision=None):
    return jax.lax.dot_general(a, b, (dims, ((), ())),
                               preferred_element_type=jnp.float32,
                               precision=precision)


_SPLIT = 2
_BH = _BM // _SPLIT


def _fused_body(x_ref, e_ref, esq_ref, w1_ref, b1_ref, w2_ref, b2_ref,
                w3_ref, b3_ref, d1w_ref, d1b_ref, d2w_ref, d2b_ref, d3w_ref,
                d3b_ref, xp_ref, zd_ref, acc_ref):
    i = pl.program_id(0)

    e = e_ref[:, :]
    esq = esq_ref[:, :]
    recon_parts = []
    vq_parts = []
    for s in range(_SPLIT):
        rows = pl.ds(s * _BH, _BH)
        x = x_ref[rows, :]
        h = _gelu_exact(_dot(x, w1_ref[:, :], ((1,), (0,))) + b1_ref[:, :])
        h = _gelu_exact(_dot(h, w2_ref[:, :], ((1,), (0,))) + b2_ref[:, :])
        z_e = _dot(h, w3_ref[:, :], ((1,), (0,))) + b3_ref[:, :]

        zsq = jnp.sum(z_e * z_e, axis=1, keepdims=True)
        cross = _dot(z_e, e, ((1,), (1,)))
        d2m = zsq + esq - 2.0 * cross
        factor = jnp.sqrt(jnp.maximum(d2m, 0.0))

        col = jax.lax.broadcasted_iota(jnp.int32, (_BH, _K), 1)
        fmin = jnp.min(factor, axis=1, keepdims=True)
        idx = jnp.min(jnp.where(factor == fmin, col, _K), axis=1,
                      keepdims=True)
        onehot = col == idx
        zd_ref[rows, :] = onehot.astype(jnp.int32)
        z_q = _dot(onehot.astype(jnp.float32), e, ((1,), (0,)),
                   precision=jax.lax.Precision.HIGHEST)

        g = _gelu_fast(_dot(z_q, d1w_ref[:, :], ((1,), (0,))) + d1b_ref[:, :])
        g = _gelu_fast(_dot(g, d2w_ref[:, :], ((1,), (0,))) + d2b_ref[:, :])
        xp = jax.nn.sigmoid(_dot(g, d3w_ref[:, :], ((1,), (0,))) +
                            d3b_ref[:, :])
        xp_ref[rows, :] = xp

        diff = x - xp
        vqd = z_e - z_q
        recon_parts.append(jnp.sum(diff * diff))
        vq_parts.append(jnp.sum(vqd * vqd))

    @pl.when(i == 0)
    def _init():
        acc_ref[0, 0] = 0.0
        acc_ref[0, 1] = 0.0

    acc_ref[0, 0] += sum(recon_parts)
    acc_ref[0, 1] += sum(vq_parts)


def kernel(x, embed_pool, W1, b1, W2, b2, W3, b3, D1, d1, D2, d2, D3, d3):
    e_sq = jnp.sum(embed_pool**2, axis=1)[None, :]

    def full(shape):
        return pl.BlockSpec(shape, lambda i: (0, 0))

    xp, zd, acc = pl.pallas_call(
        _fused_body,
        grid=(_GRID,),
        in_specs=[
            pl.BlockSpec((_BM, _IN), lambda i: (i, 0)),
            full((_K, _D)),
            full((1, _K)),
            full((_IN, _H)), full((1, _H)),
            full((_H, _H)), full((1, _H)),
            full((_H, _D)), full((1, _D)),
            full((_D, _H)), full((1, _H)),
            full((_H, _H)), full((1, _H)),
            full((_H, _IN)), full((1, _IN)),
        ],
        out_specs=[
            pl.BlockSpec((_BM, _IN), lambda i: (i, 0)),
            pl.BlockSpec((_BM, _K), lambda i: (i, 0)),
            pl.BlockSpec(memory_space=pltpu.SMEM),
        ],
        out_shape=[
            jax.ShapeDtypeStruct((_B, _IN), jnp.float32),
            jax.ShapeDtypeStruct((_B, _K), jnp.int32),
            jax.ShapeDtypeStruct((1, 2), jnp.float32),
        ],
        compiler_params=pltpu.CompilerParams(
            dimension_semantics=("arbitrary",)),
    )(x, embed_pool, e_sq, W1, b1.reshape(1, _H), W2, b2.reshape(1, _H),
      W3, b3.reshape(1, _D), D1, d1.reshape(1, _H), D2, d2.reshape(1, _H),
      D3, d3.reshape(1, _IN))

    loss = (acc[0, 0] / (_B * _IN) + 1.25 * acc[0, 1] / (_B * _D)) / _B
    return xp, zd, loss
